# baseline (speedup 1.0000x reference)
import functools

import jax
import jax.numpy as jnp
from jax import lax
from jax.experimental import pallas as pl
from jax.experimental.pallas import tpu as pltpu
from jax.experimental.pallas import tpu_sc as plsc

NUM_FIELDS = 26
VOCAB = 100000
EMB_DIM = 32
BATCH = 16384

NC, NS, L = 2, 16, 16
NW = NC * NS
N_ROWS = BATCH * NUM_FIELDS
PER_W = N_ROWS // NW
NCH = 16
CR = PER_W // NCH
NBUF = 4
LOG2_BATCH = 14

_mesh = plsc.VectorSubcoreMesh(
    core_axis_name="c", subcore_axis_name="s",
    num_cores=NC, num_subcores=NS)


@functools.partial(
    pl.kernel,
    out_type=jax.ShapeDtypeStruct((N_ROWS, EMB_DIM), jnp.float32),
    mesh=_mesh,
    compiler_params=pltpu.CompilerParams(use_tc_tiling_on_sc=False),
    scratch_types=[
        pltpu.VMEM((PER_W,), jnp.int32),
        *([pltpu.VMEM((CR, EMB_DIM), jnp.float32)] * NBUF),
        *([pltpu.SemaphoreType.DMA] * (2 * NBUF)),
    ],
)
def _embed_gather(x_hbm, tbl_hbm, out_hbm, idx_v,
                  b0, b1, b2, b3, g0, g1, g2, g3, w0, w1, w2, w3):
    bufs = (b0, b1, b2, b3)
    gsems = (g0, g1, g2, g3)
    wsems = (w0, w1, w2, w3)
    wid = lax.axis_index("s") * NC + lax.axis_index("c")
    base = wid * PER_W

    pltpu.sync_copy(x_hbm.at[pl.ds(base, PER_W)], idx_v)

    def add_offsets(s, carry):
        p0 = base + s * L
        off = (p0 >> LOG2_BATCH) * VOCAB
        sl = pl.ds(s * L, L)
        idx_v[sl] = idx_v[sl] + off
        return carry
    lax.fori_loop(0, PER_W // L, add_offsets, 0)

    def gdesc(c, b):
        return pltpu.make_async_copy(
            tbl_hbm.at[idx_v.at[pl.ds(c * CR, CR)]], bufs[b], gsems[b])

    def wdesc(c, b):
        return pltpu.make_async_copy(
            bufs[b], out_hbm.at[pl.ds(base + c * CR, CR)], wsems[b])

    for b in range(NBUF - 1):
        gdesc(b, b).start()

    def group(gi, carry):
        for b in range(NBUF):
            c = gi * NBUF + b
            gdesc(c, b).wait()
            wdesc(c, b).start()
            bb = (b + NBUF - 1) % NBUF

            @pl.when(c >= 1)
            def _():
                wdesc(c - 1, bb).wait()

            @pl.when(c + NBUF - 1 < NCH)
            def _():
                gdesc(c + NBUF - 1, bb).start()
        return carry
    lax.fori_loop(0, NCH // NBUF, group, 0)

    wdesc(NCH - 1, (NCH - 1) % NBUF).wait()


def kernel(x, tables):
    tbl = tables.reshape(NUM_FIELDS * VOCAB, EMB_DIM)
    xt = x.T.reshape(N_ROWS)
    out = _embed_gather(xt, tbl)
    return (out.reshape(NUM_FIELDS, BATCH, EMB_DIM)
            .transpose(1, 0, 2).reshape(BATCH, NUM_FIELDS * EMB_DIM))


# speedup vs baseline: 1.0705x; 1.0705x over previous
import functools

import jax
import jax.numpy as jnp
from jax import lax
from jax.experimental import pallas as pl
from jax.experimental.pallas import tpu as pltpu
from jax.experimental.pallas import tpu_sc as plsc

NUM_FIELDS = 26
VOCAB = 100000
EMB_DIM = 32
BATCH = 16384

NC, NS, L = 2, 16, 16
NW = NC * NS
N_ROWS = BATCH * NUM_FIELDS
PER_W = N_ROWS // NW
NCH = 32
CR = PER_W // NCH
NBUF = 8
PAT = 208

_mesh = plsc.VectorSubcoreMesh(
    core_axis_name="c", subcore_axis_name="s",
    num_cores=NC, num_subcores=NS)


@functools.partial(
    pl.kernel,
    out_type=jax.ShapeDtypeStruct((N_ROWS, EMB_DIM), jnp.float32),
    mesh=_mesh,
    compiler_params=pltpu.CompilerParams(use_tc_tiling_on_sc=False),
    scratch_types=[
        pltpu.VMEM((PER_W,), jnp.int32),
        pltpu.VMEM((PAT,), jnp.int32),
        *([pltpu.VMEM((CR, EMB_DIM), jnp.float32)] * NBUF),
        *([pltpu.SemaphoreType.DMA] * (2 * NBUF)),
    ],
)
def _embed_gather(x_hbm, tbl_hbm, out_hbm, idx_v, patt_v, *bufsems):
    bufs = bufsems[:NBUF]
    gsems = bufsems[NBUF:2 * NBUF]
    wsems = bufsems[2 * NBUF:]
    wid = lax.axis_index("s") * NC + lax.axis_index("c")
    base = wid * PER_W

    pltpu.sync_copy(x_hbm.at[pl.ds(base, PER_W)], idx_v)

    for j in range(PAT // L):
        lane = lax.iota(jnp.int32, L) + (j * L)
        patt_v[pl.ds(j * L, L)] = lax.rem(lane, NUM_FIELDS) * VOCAB

    def add_offsets(g, carry):
        off = g * PAT
        for j in range(PAT // L):
            sl = pl.ds(off + j * L, L)
            idx_v[sl] = idx_v[sl] + patt_v[pl.ds(j * L, L)]
        return carry
    lax.fori_loop(0, PER_W // PAT, add_offsets, 0)

    def gdesc(c, b):
        return pltpu.make_async_copy(
            tbl_hbm.at[idx_v.at[pl.ds(c * CR, CR)]], bufs[b], gsems[b])

    def wdesc(c, b):
        return pltpu.make_async_copy(
            bufs[b], out_hbm.at[pl.ds(base + c * CR, CR)], wsems[b])

    for b in range(NBUF - 1):
        gdesc(b, b).start()

    def group(gi, carry):
        for b in range(NBUF):
            c = gi * NBUF + b
            gdesc(c, b).wait()
            wdesc(c, b).start()
            bb = (b + NBUF - 1) % NBUF

            @pl.when(c >= 1)
            def _():
                wdesc(c - 1, bb).wait()

            @pl.when(c + NBUF - 1 < NCH)
            def _():
                gdesc(c + NBUF - 1, bb).start()
        return carry
    lax.fori_loop(0, NCH // NBUF, group, 0)

    wdesc(NCH - 1, (NCH - 1) % NBUF).wait()


def kernel(x, tables):
    tbl = tables.reshape(NUM_FIELDS * VOCAB, EMB_DIM)
    xf = x.reshape(N_ROWS)
    out = _embed_gather(xf, tbl)
    return out.reshape(BATCH, NUM_FIELDS * EMB_DIM)


# speedup vs baseline: 4.4698x; 4.1756x over previous
import functools

import jax
import jax.numpy as jnp
from jax import lax
from jax.experimental import pallas as pl
from jax.experimental.pallas import tpu as pltpu
from jax.experimental.pallas import tpu_sc as plsc

NUM_FIELDS = 26
VOCAB = 100000
EMB_DIM = 32
BATCH = 16384

NC, NS, L = 2, 16, 16
ROWS = NUM_FIELDS * EMB_DIM
ROWS_SC = ROWS // NC
NSB = ROWS_SC // NS
HB = BATCH // 2
UNROLL = 4

_mesh = plsc.VectorSubcoreMesh(
    core_axis_name="c", subcore_axis_name="s",
    num_cores=NC, num_subcores=NS)


@functools.partial(
    pl.kernel,
    out_type=jax.ShapeDtypeStruct((ROWS, BATCH), jnp.float32),
    mesh=_mesh,
    compiler_params=pltpu.CompilerParams(
        use_tc_tiling_on_sc=True, needs_layout_passes=False),
    scratch_types=[
        pltpu.VMEM((VOCAB,), jnp.float32),
        pltpu.VMEM((HB,), jnp.int32),
        pltpu.VMEM((HB,), jnp.float32),
        pltpu.VMEM((HB,), jnp.float32),
        pltpu.SemaphoreType.DMA,
        pltpu.SemaphoreType.DMA,
    ],
)
def _embed_gather(xt_hbm, tbl_hbm, out_hbm, row_v, idx_v, o0, o1, s0, s1):
    cid = lax.axis_index("c")
    tid = lax.axis_index("s")
    j_sc = cid * ROWS_SC
    obufs = (o0, o1)
    osems = (s0, s1)

    def super_band(k, carry):
        j = j_sc + k * NS + tid
        f = j >> 5
        pltpu.sync_copy(tbl_hbm.at[j], row_v)

        for h in range(2):
            b0 = h * HB
            pltpu.sync_copy(xt_hbm.at[f, pl.ds(b0, HB)], idx_v)
            ob = obufs[h]

            @pl.when(k > 0)
            def _():
                pltpu.make_async_copy(
                    ob, out_hbm.at[j - NS, pl.ds(b0, HB)], osems[h]).wait()

            def gather(s, c2):
                base = s * (L * UNROLL)
                for u in range(UNROLL):
                    sl = pl.ds(base + u * L, L)
                    ob[sl] = plsc.load_gather(row_v, [idx_v[sl]])
                return c2
            lax.fori_loop(0, HB // (L * UNROLL), gather, 0)
            pltpu.make_async_copy(
                ob, out_hbm.at[j, pl.ds(b0, HB)], osems[h]).start()
        return carry
    lax.fori_loop(0, NSB, super_band, 0)

    j_last = j_sc + (NSB - 1) * NS + tid
    for h in range(2):
        pltpu.make_async_copy(
            obufs[h], out_hbm.at[j_last, pl.ds(h * HB, HB)], osems[h]).wait()


def kernel(x, tables):
    tbl_t = tables.transpose(0, 2, 1).reshape(ROWS, VOCAB)
    xt = x.T
    out_t = _embed_gather(xt, tbl_t)
    return out_t.T.reshape(BATCH, NUM_FIELDS * EMB_DIM)


# speedup vs baseline: 5.4480x; 1.2188x over previous
import functools

import jax
import jax.numpy as jnp
from jax import lax
from jax.experimental import pallas as pl
from jax.experimental.pallas import tpu as pltpu
from jax.experimental.pallas import tpu_sc as plsc

NUM_FIELDS = 26
VOCAB = 100000
EMB_DIM = 32
BATCH = 16384

NC, NS, L = 2, 16, 16
ROWS = NUM_FIELDS * EMB_DIM
ROWS_SC = ROWS // NC
NSB = ROWS_SC // NS
QB = BATCH // 4
UNROLL = 8

_mesh = plsc.VectorSubcoreMesh(
    core_axis_name="c", subcore_axis_name="s",
    num_cores=NC, num_subcores=NS)


@functools.partial(
    pl.kernel,
    out_type=jax.ShapeDtypeStruct((ROWS, BATCH), jnp.float32),
    mesh=_mesh,
    compiler_params=pltpu.CompilerParams(
        use_tc_tiling_on_sc=True, needs_layout_passes=False),
    scratch_types=[
        pltpu.VMEM((VOCAB,), jnp.float32),
        pltpu.VMEM((QB,), jnp.int32),
        pltpu.VMEM((QB,), jnp.int32),
        pltpu.VMEM((QB,), jnp.float32),
        pltpu.VMEM((QB,), jnp.float32),
        pltpu.SemaphoreType.DMA,
        pltpu.SemaphoreType.DMA,
        pltpu.SemaphoreType.DMA,
        pltpu.SemaphoreType.DMA,
    ],
)
def _embed_gather(xt_hbm, tbl_hbm, out_hbm,
                  row_v, i0, i1, o0, o1, si0, si1, so0, so1):
    cid = lax.axis_index("c")
    tid = lax.axis_index("s")
    j_sc = cid * ROWS_SC
    ibufs = (i0, i1)
    isems = (si0, si1)
    obufs = (o0, o1)
    osems = (so0, so1)

    def idesc(f, q, p):
        return pltpu.make_async_copy(
            xt_hbm.at[f, pl.ds(q * QB, QB)], ibufs[p], isems[p])

    def odesc(j, q, p):
        return pltpu.make_async_copy(
            obufs[p], out_hbm.at[j, pl.ds(q * QB, QB)], osems[p])

    idesc((j_sc + tid) >> 5, 0, 0).start()

    def super_band(k, carry):
        j = j_sc + k * NS + tid
        f = j >> 5
        pltpu.sync_copy(tbl_hbm.at[j], row_v)

        for q in range(4):
            p = q & 1
            idesc(f, q, p).wait()
            ob = obufs[p]

            @pl.when(jnp.logical_or(k > 0, q >= 2))
            def _():
                odesc(j, q, p).wait()

            if q < 3:
                idesc(f, q + 1, (q + 1) & 1).start()
            else:
                @pl.when(k < NSB - 1)
                def _():
                    idesc((j + NS) >> 5, 0, 0).start()

            iv = ibufs[p]

            def gather(s, c2):
                base = s * (L * UNROLL)
                for u in range(UNROLL):
                    sl = pl.ds(base + u * L, L)
                    ob[sl] = plsc.load_gather(row_v, [iv[sl]])
                return c2
            lax.fori_loop(0, QB // (L * UNROLL), gather, 0)
            odesc(j, q, p).start()
        return carry
    lax.fori_loop(0, NSB, super_band, 0)

    j_last = j_sc + (NSB - 1) * NS + tid
    for q in (2, 3):
        odesc(j_last, q, q & 1).wait()


def kernel(x, tables):
    tbl_t = tables.transpose(0, 2, 1).reshape(ROWS, VOCAB)
    xt = x.T
    out_t = _embed_gather(xt, tbl_t)
    return out_t.T.reshape(BATCH, NUM_FIELDS * EMB_DIM)


# speedup vs baseline: 5.4648x; 1.0031x over previous
import functools

import jax
import jax.numpy as jnp
from jax import lax
from jax.experimental import pallas as pl
from jax.experimental.pallas import tpu as pltpu
from jax.experimental.pallas import tpu_sc as plsc

NUM_FIELDS = 26
VOCAB = 100000
EMB_DIM = 32
BATCH = 16384

NC, NS, L = 2, 16, 16
ROWS = NUM_FIELDS * EMB_DIM
ROWS_SC = ROWS // NC
NSB = ROWS_SC // NS
QB = BATCH // 4
UNROLL = 16

_mesh = plsc.VectorSubcoreMesh(
    core_axis_name="c", subcore_axis_name="s",
    num_cores=NC, num_subcores=NS)


@functools.partial(
    pl.kernel,
    out_type=jax.ShapeDtypeStruct((ROWS, BATCH), jnp.float32),
    mesh=_mesh,
    compiler_params=pltpu.CompilerParams(
        use_tc_tiling_on_sc=True, needs_layout_passes=False),
    scratch_types=[
        pltpu.VMEM((VOCAB,), jnp.float32),
        pltpu.VMEM((QB,), jnp.int32),
        pltpu.VMEM((QB,), jnp.int32),
        pltpu.VMEM((QB,), jnp.float32),
        pltpu.VMEM((QB,), jnp.float32),
        pltpu.SemaphoreType.DMA,
        pltpu.SemaphoreType.DMA,
        pltpu.SemaphoreType.DMA,
        pltpu.SemaphoreType.DMA,
    ],
)
def _embed_gather(xt_hbm, tbl_hbm, out_hbm,
                  row_v, i0, i1, o0, o1, si0, si1, so0, so1):
    cid = lax.axis_index("c")
    tid = lax.axis_index("s")
    j_sc = cid * ROWS_SC
    ibufs = (i0, i1)
    isems = (si0, si1)
    obufs = (o0, o1)
    osems = (so0, so1)

    def idesc(f, q, p):
        return pltpu.make_async_copy(
            xt_hbm.at[f, pl.ds(q * QB, QB)], ibufs[p], isems[p])

    def odesc(j, q, p):
        return pltpu.make_async_copy(
            obufs[p], out_hbm.at[j, pl.ds(q * QB, QB)], osems[p])

    idesc((j_sc + tid) >> 5, 0, 0).start()

    def super_band(k, carry):
        j = j_sc + k * NS + tid
        f = j >> 5
        pltpu.sync_copy(tbl_hbm.at[j], row_v)

        for q in range(4):
            p = q & 1
            idesc(f, q, p).wait()
            ob = obufs[p]

            @pl.when(jnp.logical_or(k > 0, q >= 2))
            def _():
                odesc(j, q, p).wait()

            if q < 3:
                idesc(f, q + 1, (q + 1) & 1).start()
            else:
                @pl.when(k < NSB - 1)
                def _():
                    idesc((j + NS) >> 5, 0, 0).start()

            iv = ibufs[p]

            def gather(s, c2):
                base = s * (L * UNROLL)
                for u in range(UNROLL):
                    sl = pl.ds(base + u * L, L)
                    ob[sl] = plsc.load_gather(row_v, [iv[sl]])
                return c2
            lax.fori_loop(0, QB // (L * UNROLL), gather, 0)
            odesc(j, q, p).start()
        return carry
    lax.fori_loop(0, NSB, super_band, 0)

    j_last = j_sc + (NSB - 1) * NS + tid
    for q in (2, 3):
        odesc(j_last, q, q & 1).wait()


def kernel(x, tables):
    tbl_t = tables.transpose(0, 2, 1).reshape(ROWS, VOCAB)
    xt = x.T
    out_t = _embed_gather(xt, tbl_t)
    return out_t.T.reshape(BATCH, NUM_FIELDS * EMB_DIM)
